# baseline (speedup 1.0000x reference)
import functools

import jax
import jax.numpy as jnp
from jax import lax
from jax.experimental import pallas as pl
from jax.experimental.pallas import tpu as pltpu
from jax.experimental.pallas import tpu_sc as plsc

B = 4
S = 2048
D = 2048
FLAT = B * S
NC = 2
NS = 16
NW = NC * NS
PW = S // NW
CH = 8
NE = PW // CH
ROWS_PER_W = PW * B
NCH = ROWS_PER_W // CH
NT = 4
LANES = 16


def _body(xf_hbm, tok_hbm, pos_hbm, out_hbm,
          idx_v, tok_v, pos_res, sem_tok, sem_pos, sem_out):
    wid = lax.axis_index("s") * NC + lax.axis_index("c")
    pos0 = wid * PW

    for bb in range(B):
        pltpu.sync_copy(xf_hbm.at[pl.ds(bb * S + pos0, PW)],
                        idx_v.at[pl.ds(bb * PW, PW)])

    def pos_copy(e, pb):
        return pltpu.make_async_copy(
            pos_hbm.at[pl.ds(pos0 + e * CH, CH)], pos_res.at[pb], sem_pos[pb])

    def gather_copy(c, t):
        e = c // B
        b = c % B
        return pltpu.make_async_copy(
            tok_hbm.at[idx_v.at[pl.ds(b * PW + e * CH, CH)]],
            tok_v.at[t], sem_tok[t])

    def out_copy(c, t):
        e = c // B
        b = c % B
        return pltpu.make_async_copy(
            tok_v.at[t], out_hbm.at[pl.ds(b * S + pos0 + e * CH, CH)],
            sem_out[t])

    def add_chunk(t, pb):
        def vbody(i, _):
            s0 = i * LANES
            for k in range(CH):
                plsc.addupdate(tok_v.at[t, k, pl.ds(s0, LANES)],
                               pos_res[pb, k, pl.ds(s0, LANES)])
            return 0
        lax.fori_loop(0, D // LANES, vbody, 0)

    pos_copy(0, 0).start()
    pos_copy(1, 1).start()
    gather_copy(0, 0).start()
    gather_copy(1, 1).start()

    def step(g, _):
        for u in range(2 * B):
            c = g * (2 * B) + u
            t = u % NT
            pb = u // B
            e = g * 2 + pb

            if u % B == 0:
                pos_copy(e, pb).wait()

            gather_copy(c, t).wait()

            if u >= 2:
                out_copy(c - 2, (u - 2) % NT).wait()
            else:
                @pl.when(c >= 2)
                def _():
                    out_copy(c - 2, (u - 2) % NT).wait()

            add_chunk(t, pb)

            out_copy(c, t).start()

            if u < 2 * B - 2:
                gather_copy(c + 2, (u + 2) % NT).start()
            else:
                @pl.when(c + 2 < NCH)
                def _():
                    gather_copy(c + 2, (u + 2) % NT).start()

            if u % B == B - 1:
                @pl.when(e + 2 < NE)
                def _():
                    pos_copy(e + 2, pb).start()
        return 0

    lax.fori_loop(0, NCH // (2 * B), step, 0)

    out_copy(NCH - 2, (NCH - 2) % NT).wait()
    out_copy(NCH - 1, (NCH - 1) % NT).wait()


def _run(xf, token_emb, pos_emb):
    mesh = plsc.VectorSubcoreMesh(core_axis_name="c", subcore_axis_name="s")
    kern = functools.partial(
        pl.kernel,
        mesh=mesh,
        out_type=jax.ShapeDtypeStruct((FLAT, D), jnp.float32),
        scratch_types=[
            pltpu.VMEM((ROWS_PER_W,), jnp.int32),
            pltpu.VMEM((NT, CH, D), jnp.float32),
            pltpu.VMEM((2, CH, D), jnp.float32),
            [pltpu.SemaphoreType.DMA] * NT,
            [pltpu.SemaphoreType.DMA] * 2,
            [pltpu.SemaphoreType.DMA] * NT,
        ],
    )(_body)
    return kern(xf, token_emb, pos_emb)


def kernel(x, token_emb, pos_emb):
    xf = x.astype(jnp.int32).reshape(FLAT)
    out = _run(xf, token_emb, pos_emb)
    return out.reshape(B, S, D)


# speedup vs baseline: 1.7521x; 1.7521x over previous
import functools

import jax
import jax.numpy as jnp
from jax import lax
from jax.experimental import pallas as pl
from jax.experimental.pallas import tpu as pltpu
from jax.experimental.pallas import tpu_sc as plsc

B = 4
S = 2048
D = 2048
FLAT = B * S
NC = 2
NS = 16
NW = NC * NS
PW = S // NW
CH = 8
NE = PW // CH
ROWS_PER_W = PW * B
NCH = ROWS_PER_W // CH
NBUF = 2
LANES = 16


def _body(xf_hbm, tok_hbm, pos_hbm, out_hbm,
          idx_v, tok_v, pos_res, out_v, sem_tok, sem_pos, sem_out):
    wid = lax.axis_index("s") * NC + lax.axis_index("c")
    pos0 = wid * PW

    for bb in range(B):
        pltpu.sync_copy(xf_hbm.at[pl.ds(bb * S + pos0, PW)],
                        idx_v.at[pl.ds(bb * PW, PW)])

    def pos_copy(e, pb):
        return pltpu.make_async_copy(
            pos_hbm.at[pl.ds(pos0 + e * CH, CH)], pos_res.at[pb], sem_pos[pb])

    def gather_copy(c, t):
        e = c // B
        b = c % B
        return pltpu.make_async_copy(
            tok_hbm.at[idx_v.at[pl.ds(b * PW + e * CH, CH)]],
            tok_v.at[t], sem_tok[t])

    def out_copy(c, t):
        e = c // B
        b = c % B
        return pltpu.make_async_copy(
            out_v.at[t], out_hbm.at[pl.ds(b * S + pos0 + e * CH, CH)],
            sem_out[t])

    def add_chunk(t, pb):
        @functools.partial(plsc.parallel_loop, 0, D // LANES, unroll=4)
        def _(i):
            s0 = i * LANES
            for k in range(CH):
                out_v[t, k, pl.ds(s0, LANES)] = (
                    tok_v[t, k, pl.ds(s0, LANES)]
                    + pos_res[pb, k, pl.ds(s0, LANES)])

    pos_copy(0, 0).start()
    pos_copy(1, 1).start()
    gather_copy(0, 0).start()
    gather_copy(1, 1).start()

    def step(g, _):
        for u in range(2 * B):
            c = g * (2 * B) + u
            t = u % NBUF
            pb = u // B
            e = g * 2 + pb

            if u % B == 0:
                pos_copy(e, pb).wait()

            gather_copy(c, t).wait()

            @pl.when(c >= NBUF)
            def _():
                out_copy(c - NBUF, t).wait()

            add_chunk(t, pb)

            @pl.when(c + NBUF < NCH)
            def _():
                gather_copy(c + NBUF, t).start()

            out_copy(c, t).start()

            if u % B == B - 1:
                @pl.when(e + 2 < NE)
                def _():
                    pos_copy(e + 2, pb).start()
        return 0

    lax.fori_loop(0, NCH // (2 * B), step, 0)

    for t in range(NBUF):
        out_copy(NCH - NBUF + t, t).wait()


def _run(xf, token_emb, pos_emb):
    mesh = plsc.VectorSubcoreMesh(core_axis_name="c", subcore_axis_name="s")
    kern = functools.partial(
        pl.kernel,
        mesh=mesh,
        out_type=jax.ShapeDtypeStruct((FLAT, D), jnp.float32),
        scratch_types=[
            pltpu.VMEM((ROWS_PER_W,), jnp.int32),
            pltpu.VMEM((NBUF, CH, D), jnp.float32),
            pltpu.VMEM((2, CH, D), jnp.float32),
            pltpu.VMEM((NBUF, CH, D), jnp.float32),
            [pltpu.SemaphoreType.DMA] * NBUF,
            [pltpu.SemaphoreType.DMA] * 2,
            [pltpu.SemaphoreType.DMA] * NBUF,
        ],
    )(_body)
    return kern(xf, token_emb, pos_emb)


def kernel(x, token_emb, pos_emb):
    xf = x.astype(jnp.int32).reshape(FLAT)
    out = _run(xf, token_emb, pos_emb)
    return out.reshape(B, S, D)


# speedup vs baseline: 1.7540x; 1.0011x over previous
import functools

import jax
import jax.numpy as jnp
from jax import lax
from jax.experimental import pallas as pl
from jax.experimental.pallas import tpu as pltpu
from jax.experimental.pallas import tpu_sc as plsc

B = 4
S = 2048
D = 2048
FLAT = B * S
NC = 2
NS = 16
NW = NC * NS
PW = S // NW
CH = 8
NE = PW // CH
ROWS_PER_W = PW * B
NCH = ROWS_PER_W // CH
NT = 4
LANES = 16


def _body(xf_hbm, tok_hbm, pos_hbm, out_hbm,
          idx_v, tok_v, pos_res, sem_tok, sem_pos, sem_out):
    wid = lax.axis_index("s") * NC + lax.axis_index("c")
    pos0 = wid * PW

    for bb in range(B):
        pltpu.sync_copy(xf_hbm.at[pl.ds(bb * S + pos0, PW)],
                        idx_v.at[pl.ds(bb * PW, PW)])

    def pos_copy(e, pb):
        return pltpu.make_async_copy(
            pos_hbm.at[pl.ds(pos0 + e * CH, CH)], pos_res.at[pb], sem_pos[pb])

    def gather_copy(c, t):
        e = c // B
        b = c % B
        return pltpu.make_async_copy(
            tok_hbm.at[idx_v.at[pl.ds(b * PW + e * CH, CH)]],
            tok_v.at[t], sem_tok[t])

    def out_copy(c, t):
        e = c // B
        b = c % B
        return pltpu.make_async_copy(
            tok_v.at[t], out_hbm.at[pl.ds(b * S + pos0 + e * CH, CH)],
            sem_out[t])

    def add_chunk(t, pb):
        @functools.partial(plsc.parallel_loop, 0, D // LANES, unroll=4)
        def _(i):
            s0 = i * LANES
            for k in range(CH):
                tok_v[t, k, pl.ds(s0, LANES)] = (
                    tok_v[t, k, pl.ds(s0, LANES)]
                    + pos_res[pb, k, pl.ds(s0, LANES)])

    pos_copy(0, 0).start()
    pos_copy(1, 1).start()
    gather_copy(0, 0).start()
    gather_copy(1, 1).start()

    def step(g, _):
        for u in range(2 * B):
            c = g * (2 * B) + u
            t = u % NT
            pb = u // B
            e = g * 2 + pb

            if u % B == 0:
                pos_copy(e, pb).wait()

            gather_copy(c, t).wait()

            if u >= 2:
                out_copy(c - 2, (u - 2) % NT).wait()
            else:
                @pl.when(c >= 2)
                def _():
                    out_copy(c - 2, (u - 2) % NT).wait()

            add_chunk(t, pb)

            if u < 2 * B - 2:
                gather_copy(c + 2, (u + 2) % NT).start()
            else:
                @pl.when(c + 2 < NCH)
                def _():
                    gather_copy(c + 2, (u + 2) % NT).start()

            out_copy(c, t).start()

            if u % B == B - 1:
                @pl.when(e + 2 < NE)
                def _():
                    pos_copy(e + 2, pb).start()
        return 0

    lax.fori_loop(0, NCH // (2 * B), step, 0)

    out_copy(NCH - 2, (NCH - 2) % NT).wait()
    out_copy(NCH - 1, (NCH - 1) % NT).wait()


def _run(xf, token_emb, pos_emb):
    mesh = plsc.VectorSubcoreMesh(core_axis_name="c", subcore_axis_name="s")
    kern = functools.partial(
        pl.kernel,
        mesh=mesh,
        out_type=jax.ShapeDtypeStruct((FLAT, D), jnp.float32),
        scratch_types=[
            pltpu.VMEM((ROWS_PER_W,), jnp.int32),
            pltpu.VMEM((NT, CH, D), jnp.float32),
            pltpu.VMEM((2, CH, D), jnp.float32),
            [pltpu.SemaphoreType.DMA] * NT,
            [pltpu.SemaphoreType.DMA] * 2,
            [pltpu.SemaphoreType.DMA] * NT,
        ],
    )(_body)
    return kern(xf, token_emb, pos_emb)


def kernel(x, token_emb, pos_emb):
    xf = x.astype(jnp.int32).reshape(FLAT)
    out = _run(xf, token_emb, pos_emb)
    return out.reshape(B, S, D)
